# baseline (speedup 1.0000x reference)
import dataclasses
import functools

import jax
import jax.numpy as jnp
from jax import lax
from jax.experimental import pallas as pl
from jax.experimental.pallas import tpu as pltpu
from jax.experimental.pallas import tpu_sc as plsc

N = 10000
D = 128
R = 10240
E = 320000
NC, NS = 2, 16
NW = NC * NS
CH = 128
NCHUNK = 80
E_W = NCHUNK * CH
E_PAD = NW * E_W
ROWS_SUB = R // NS
HR = R // 128

f32 = jnp.float32
i32 = jnp.int32

_mesh = plsc.VectorSubcoreMesh(core_axis_name="c", subcore_axis_name="s")
_cp = pltpu.CompilerParams()
if "needs_layout_passes" in pltpu.CompilerParams.__dataclass_fields__:
    _cp = dataclasses.replace(_cp, needs_layout_passes=False)


@functools.partial(
    pl.kernel,
    out_type=jax.ShapeDtypeStruct((NW * HR, 128), f32),
    mesh=_mesh,
    compiler_params=_cp,
    scratch_types=[
        pltpu.VMEM((E_W,), i32),
        pltpu.VMEM((HR, 128), f32),
    ],
)
def _deg_kernel(dst_hbm, o_hbm, idx_v, deg_v):
    c = lax.axis_index("c")
    s = lax.axis_index("s")
    wid = c * NS + s

    @pl.loop(0, HR)
    def _(r):
        for k in range(8):
            deg_v[r, pl.ds(k * 16, 16)] = jnp.zeros((16,), f32)

    pltpu.sync_copy(dst_hbm.at[pl.ds(pl.multiple_of(wid * E_W, 128), E_W)],
                    idx_v)

    @pl.loop(0, E_W // 16)
    def _(k):
        idx16 = idx_v[pl.ds(k * 16, 16)]
        plsc.addupdate_scatter(deg_v, [idx16 >> 7, idx16 & 127],
                               jnp.ones((16,), f32))

    pltpu.sync_copy(deg_v, o_hbm.at[pl.ds(pl.multiple_of(wid * HR, 8), HR)])


@functools.partial(
    pl.kernel,
    out_type=jax.ShapeDtypeStruct((NC, R, D), f32),
    mesh=_mesh,
    scratch_types=[
        pltpu.VMEM((CH,), i32),
        pltpu.VMEM((CH,), i32),
        pltpu.VMEM((CH, D), f32),
        pltpu.VMEM_SHARED((R, D), f32),
    ],
)
def _agg_kernel(hs_hbm, src_hbm, dst_hbm, p_hbm,
                src_v, dst_v, rows_v, acc):
    c = lax.axis_index("c")
    s = lax.axis_index("s")
    wid = c * NS + s
    r0 = pl.multiple_of(s * ROWS_SUB, 8)

    @pl.loop(0, CH)
    def _(r):
        for k in range(D // 16):
            rows_v[r, pl.ds(k * 16, 16)] = jnp.zeros((16,), f32)

    @pl.loop(0, ROWS_SUB // CH)
    def _(t):
        pltpu.sync_copy(rows_v, acc.at[pl.ds(r0 + t * CH, CH)])

    plsc.subcore_barrier()
    base = wid * E_W

    @pl.loop(0, NCHUNK)
    def _(j):
        off = base + j * CH
        pltpu.sync_copy(src_hbm.at[pl.ds(off, CH)], src_v)
        pltpu.sync_copy(dst_hbm.at[pl.ds(off, CH)], dst_v)
        pltpu.sync_copy(hs_hbm.at[src_v], rows_v)
        pltpu.sync_copy(rows_v, acc.at[dst_v], add=True)

    plsc.subcore_barrier()
    pltpu.sync_copy(acc.at[pl.ds(r0, ROWS_SUB)],
                    p_hbm.at[c].at[pl.ds(r0, ROWS_SUB)])


BM = 640
BN = 400
_HI = lax.Precision.HIGHEST


def _mm_body(x_ref, w_ref, o_ref):
    o_ref[...] = jnp.dot(x_ref[...], w_ref[...],
                         preferred_element_type=f32, precision=_HI)


_mm = pl.pallas_call(
    _mm_body,
    grid=(R // BM,),
    in_specs=[pl.BlockSpec((BM, D), lambda i: (i, 0)),
              pl.BlockSpec((D, D), lambda i: (0, 0))],
    out_specs=pl.BlockSpec((BM, D), lambda i: (i, 0)),
    out_shape=jax.ShapeDtypeStruct((R, D), f32),
)


def _scale_body(h_ref, d_ref, o_ref):
    o_ref[...] = h_ref[...] * lax.rsqrt(d_ref[...])


_scale = pl.pallas_call(
    _scale_body,
    grid=(R // BM,),
    in_specs=[pl.BlockSpec((BM, D), lambda i: (i, 0)),
              pl.BlockSpec((BM, 1), lambda i: (i, 0))],
    out_specs=pl.BlockSpec((BM, D), lambda i: (i, 0)),
    out_shape=jax.ShapeDtypeStruct((R, D), f32),
)


def _combine1_body(p0_ref, p1_ref, hs_ref, d_ref, b_ref, w_ref,
                   h_ref, hs2_ref):
    dis = lax.rsqrt(d_ref[...])
    agg = p0_ref[0] + p1_ref[0] + hs_ref[...]
    e = jnp.maximum(dis * agg + b_ref[...], 0.0)
    h_ref[...] = e
    hs2_ref[...] = jnp.dot(e, w_ref[...],
                           preferred_element_type=f32, precision=_HI) * dis


_combine1 = pl.pallas_call(
    _combine1_body,
    grid=(R // BM,),
    in_specs=[pl.BlockSpec((1, BM, D), lambda i: (0, i, 0)),
              pl.BlockSpec((1, BM, D), lambda i: (1, i, 0)),
              pl.BlockSpec((BM, D), lambda i: (i, 0)),
              pl.BlockSpec((BM, 1), lambda i: (i, 0)),
              pl.BlockSpec((1, D), lambda i: (0, 0)),
              pl.BlockSpec((D, D), lambda i: (0, 0))],
    out_specs=[pl.BlockSpec((BM, D), lambda i: (i, 0)),
               pl.BlockSpec((BM, D), lambda i: (i, 0))],
    out_shape=[jax.ShapeDtypeStruct((R, D), f32),
               jax.ShapeDtypeStruct((R, D), f32)],
)


def _combine2_body(q0_ref, q1_ref, hs_ref, d_ref, b_ref, h1_ref, o_ref):
    dis = lax.rsqrt(d_ref[...])
    agg = q0_ref[0] + q1_ref[0] + hs_ref[...]
    h2 = jnp.maximum(dis * agg + b_ref[...], 0.0)
    o_ref[...] = jnp.concatenate([h1_ref[...], h2], axis=1)


_combine2 = pl.pallas_call(
    _combine2_body,
    grid=(N // BN,),
    in_specs=[pl.BlockSpec((1, BN, D), lambda i: (0, i, 0)),
              pl.BlockSpec((1, BN, D), lambda i: (1, i, 0)),
              pl.BlockSpec((BN, D), lambda i: (i, 0)),
              pl.BlockSpec((BN, 1), lambda i: (i, 0)),
              pl.BlockSpec((1, D), lambda i: (0, 0)),
              pl.BlockSpec((BN, D), lambda i: (i, 0))],
    out_specs=pl.BlockSpec((BN, 2 * D), lambda i: (i, 0)),
    out_shape=jax.ShapeDtypeStruct((N, 2 * D), f32),
)


def kernel(x, edge_index, W1, b1, W2, b2):
    x = x.astype(f32)
    src = edge_index[0].astype(i32)
    dst = edge_index[1].astype(i32)
    pad_idx = jnp.full((E_PAD - E,), N, i32)
    src_p = jnp.concatenate([src, pad_idx])
    dst_p = jnp.concatenate([dst, pad_idx])
    x_pad = jnp.concatenate([x, jnp.zeros((R - N, D), f32)])
    b1r = b1.reshape(1, D).astype(f32)
    b2r = b2.reshape(1, D).astype(f32)

    deg32 = _deg_kernel(dst_p)
    degp = deg32.reshape(NW, R).sum(axis=0).reshape(R, 1) + 1.0
    h1 = _mm(x_pad, W1.astype(f32))
    hs1 = _scale(h1, degp)
    p = _agg_kernel(hs1, src_p, dst_p)
    h1o, hs2 = _combine1(p, p, hs1, degp, b1r, W2.astype(f32))
    q = _agg_kernel(hs2, src_p, dst_p)
    return _combine2(q, q, hs2, degp, b2r, h1o)


# speedup vs baseline: 1.3767x; 1.3767x over previous
import dataclasses
import functools

import jax
import jax.numpy as jnp
from jax import lax
from jax.experimental import pallas as pl
from jax.experimental.pallas import tpu as pltpu
from jax.experimental.pallas import tpu_sc as plsc

N = 10000
D = 128
R = 10240
E = 320000
NC, NS = 2, 16
NW = NC * NS
CH = 128
NCHUNK = 79
E_W = NCHUNK * CH
E_PAD = NW * E_W
ROWS_SUB = R // NS
HR = R // 128

f32 = jnp.float32
i32 = jnp.int32

_mesh = plsc.VectorSubcoreMesh(core_axis_name="c", subcore_axis_name="s")
_cp = pltpu.CompilerParams()
if "needs_layout_passes" in pltpu.CompilerParams.__dataclass_fields__:
    _cp = dataclasses.replace(_cp, needs_layout_passes=False)


@functools.partial(
    pl.kernel,
    out_type=jax.ShapeDtypeStruct((NW * HR, 128), f32),
    mesh=_mesh,
    compiler_params=_cp,
    scratch_types=[
        pltpu.VMEM((E_W,), i32),
        pltpu.VMEM((HR, 128), f32),
    ],
)
def _deg_kernel(dst_hbm, o_hbm, idx_v, deg_v):
    c = lax.axis_index("c")
    s = lax.axis_index("s")
    wid = c * NS + s

    @pl.loop(0, HR)
    def _(r):
        for k in range(8):
            deg_v[r, pl.ds(k * 16, 16)] = jnp.zeros((16,), f32)

    pltpu.sync_copy(dst_hbm.at[pl.ds(pl.multiple_of(wid * E_W, 128), E_W)],
                    idx_v)

    @pl.loop(0, E_W // 16)
    def _(k):
        idx16 = idx_v[pl.ds(k * 16, 16)]
        plsc.addupdate_scatter(deg_v, [idx16 >> 7, idx16 & 127],
                               jnp.ones((16,), f32))

    pltpu.sync_copy(deg_v, o_hbm.at[pl.ds(pl.multiple_of(wid * HR, 8), HR)])


@functools.partial(
    pl.kernel,
    out_type=jax.ShapeDtypeStruct((NC, R, D), f32),
    mesh=_mesh,
    scratch_types=[
        pltpu.VMEM((CH,), i32),
        pltpu.VMEM((CH,), i32),
        pltpu.VMEM((CH, D), f32),
        pltpu.VMEM_SHARED((R, D), f32),
    ],
)
def _agg_kernel(hs_hbm, src_hbm, dst_hbm, p_hbm,
                src_v, dst_v, rows_v, acc):
    c = lax.axis_index("c")
    s = lax.axis_index("s")
    wid = c * NS + s
    r0 = pl.multiple_of(s * ROWS_SUB, 8)

    @pl.loop(0, CH)
    def _(r):
        for k in range(D // 16):
            rows_v[r, pl.ds(k * 16, 16)] = jnp.zeros((16,), f32)

    @pl.loop(0, ROWS_SUB // CH)
    def _(t):
        pltpu.sync_copy(rows_v, acc.at[pl.ds(r0 + t * CH, CH)])

    plsc.subcore_barrier()
    base = wid * E_W

    @pl.loop(0, NCHUNK)
    def _(j):
        off = base + j * CH
        pltpu.sync_copy(src_hbm.at[pl.ds(off, CH)], src_v)
        pltpu.sync_copy(dst_hbm.at[pl.ds(off, CH)], dst_v)
        pltpu.sync_copy(hs_hbm.at[src_v], rows_v)
        pltpu.sync_copy(rows_v, acc.at[dst_v], add=True)

    plsc.subcore_barrier()
    pltpu.sync_copy(acc.at[pl.ds(r0, ROWS_SUB)],
                    p_hbm.at[c].at[pl.ds(r0, ROWS_SUB)])


BM = 640
BN = 400
_HI = lax.Precision.HIGHEST


def _mm_body(x_ref, w_ref, o_ref):
    o_ref[...] = jnp.dot(x_ref[...], w_ref[...],
                         preferred_element_type=f32, precision=_HI)


_mm = pl.pallas_call(
    _mm_body,
    grid=(R // BM,),
    in_specs=[pl.BlockSpec((BM, D), lambda i: (i, 0)),
              pl.BlockSpec((D, D), lambda i: (0, 0))],
    out_specs=pl.BlockSpec((BM, D), lambda i: (i, 0)),
    out_shape=jax.ShapeDtypeStruct((R, D), f32),
)


def _scale_body(h_ref, d_ref, o_ref):
    o_ref[...] = h_ref[...] * lax.rsqrt(d_ref[...])


_scale = pl.pallas_call(
    _scale_body,
    grid=(R // BM,),
    in_specs=[pl.BlockSpec((BM, D), lambda i: (i, 0)),
              pl.BlockSpec((BM, 1), lambda i: (i, 0))],
    out_specs=pl.BlockSpec((BM, D), lambda i: (i, 0)),
    out_shape=jax.ShapeDtypeStruct((R, D), f32),
)


def _combine1_body(p0_ref, p1_ref, hs_ref, d_ref, b_ref, w_ref,
                   h_ref, hs2_ref):
    dis = lax.rsqrt(d_ref[...])
    agg = p0_ref[0] + p1_ref[0] + hs_ref[...]
    e = jnp.maximum(dis * agg + b_ref[...], 0.0)
    h_ref[...] = e
    hs2_ref[...] = jnp.dot(e, w_ref[...],
                           preferred_element_type=f32, precision=_HI) * dis


_combine1 = pl.pallas_call(
    _combine1_body,
    grid=(R // BM,),
    in_specs=[pl.BlockSpec((1, BM, D), lambda i: (0, i, 0)),
              pl.BlockSpec((1, BM, D), lambda i: (1, i, 0)),
              pl.BlockSpec((BM, D), lambda i: (i, 0)),
              pl.BlockSpec((BM, 1), lambda i: (i, 0)),
              pl.BlockSpec((1, D), lambda i: (0, 0)),
              pl.BlockSpec((D, D), lambda i: (0, 0))],
    out_specs=[pl.BlockSpec((BM, D), lambda i: (i, 0)),
               pl.BlockSpec((BM, D), lambda i: (i, 0))],
    out_shape=[jax.ShapeDtypeStruct((R, D), f32),
               jax.ShapeDtypeStruct((R, D), f32)],
)


def _combine2_body(q0_ref, q1_ref, hs_ref, d_ref, b_ref, h1_ref, o_ref):
    dis = lax.rsqrt(d_ref[...])
    agg = q0_ref[0] + q1_ref[0] + hs_ref[...]
    h2 = jnp.maximum(dis * agg + b_ref[...], 0.0)
    o_ref[...] = jnp.concatenate([h1_ref[...], h2], axis=1)


_combine2 = pl.pallas_call(
    _combine2_body,
    grid=(N // BN,),
    in_specs=[pl.BlockSpec((1, BN, D), lambda i: (0, i, 0)),
              pl.BlockSpec((1, BN, D), lambda i: (1, i, 0)),
              pl.BlockSpec((BN, D), lambda i: (i, 0)),
              pl.BlockSpec((BN, 1), lambda i: (i, 0)),
              pl.BlockSpec((1, D), lambda i: (0, 0)),
              pl.BlockSpec((BN, D), lambda i: (i, 0))],
    out_specs=pl.BlockSpec((BN, 2 * D), lambda i: (i, 0)),
    out_shape=jax.ShapeDtypeStruct((N, 2 * D), f32),
)


def kernel(x, edge_index, W1, b1, W2, b2):
    x = x.astype(f32)
    src = edge_index[0].astype(i32)
    dst = edge_index[1].astype(i32)
    pad_idx = jnp.full((E_PAD - E,), N, i32)
    src_p = jnp.concatenate([src, pad_idx])
    dst_p = jnp.concatenate([dst, pad_idx])
    x_pad = jnp.concatenate([x, jnp.zeros((R - N, D), f32)])
    b1r = b1.reshape(1, D).astype(f32)
    b2r = b2.reshape(1, D).astype(f32)

    deg32 = _deg_kernel(dst_p)
    degp = deg32.reshape(NW, R).sum(axis=0).reshape(R, 1) + 1.0
    h1 = _mm(x_pad, W1.astype(f32))
    hs1 = _scale(h1, degp)
    p = _agg_kernel(hs1, src_p, dst_p)
    h1o, hs2 = _combine1(p, p, hs1, degp, b1r, W2.astype(f32))
    q = _agg_kernel(hs2, src_p, dst_p)
    return _combine2(q, q, hs2, degp, b2r, h1o)


# speedup vs baseline: 1.4547x; 1.0567x over previous
import dataclasses
import functools

import jax
import jax.numpy as jnp
from jax import lax
from jax.experimental import pallas as pl
from jax.experimental.pallas import tpu as pltpu
from jax.experimental.pallas import tpu_sc as plsc

N = 10000
D = 128
R = 10240
E = 320000
NC, NS = 2, 16
NW = NC * NS
CH = 128
NCHUNK = 79
E_W = NCHUNK * CH
E_PAD = NW * E_W
ROWS_SUB = R // NS
HR = R // 128

f32 = jnp.float32
i32 = jnp.int32

_mesh = plsc.VectorSubcoreMesh(core_axis_name="c", subcore_axis_name="s")
_cp = pltpu.CompilerParams()
if "needs_layout_passes" in pltpu.CompilerParams.__dataclass_fields__:
    _cp = dataclasses.replace(_cp, needs_layout_passes=False)


@functools.partial(
    pl.kernel,
    out_type=jax.ShapeDtypeStruct((NW * HR, 128), f32),
    mesh=_mesh,
    compiler_params=_cp,
    scratch_types=[
        pltpu.VMEM((E_W,), i32),
        pltpu.VMEM((HR, 128), f32),
    ],
)
def _deg_kernel(dst_hbm, o_hbm, idx_v, deg_v):
    c = lax.axis_index("c")
    s = lax.axis_index("s")
    wid = c * NS + s

    @pl.loop(0, HR)
    def _(r):
        for k in range(8):
            deg_v[r, pl.ds(k * 16, 16)] = jnp.zeros((16,), f32)

    pltpu.sync_copy(dst_hbm.at[pl.ds(pl.multiple_of(wid * E_W, 128), E_W)],
                    idx_v)

    @pl.loop(0, E_W // 16)
    def _(k):
        idx16 = idx_v[pl.ds(k * 16, 16)]
        plsc.addupdate_scatter(deg_v, [idx16 >> 7, idx16 & 127],
                               jnp.ones((16,), f32))

    pltpu.sync_copy(deg_v, o_hbm.at[pl.ds(pl.multiple_of(wid * HR, 8), HR)])


@functools.partial(
    pl.kernel,
    out_type=jax.ShapeDtypeStruct((NC, R, D), f32),
    mesh=_mesh,
    scratch_types=[
        pltpu.VMEM((CH,), i32),
        pltpu.VMEM((CH,), i32),
        pltpu.VMEM((CH, D), f32),
        pltpu.VMEM_SHARED((R, D), f32),
    ],
)
def _agg_kernel(hs_hbm, src_hbm, dst_hbm, p_hbm,
                src_v, dst_v, rows_v, acc):
    c = lax.axis_index("c")
    s = lax.axis_index("s")
    wid = c * NS + s
    r0 = pl.multiple_of(s * ROWS_SUB, 8)

    @pl.loop(0, CH)
    def _(r):
        for k in range(D // 16):
            rows_v[r, pl.ds(k * 16, 16)] = jnp.zeros((16,), f32)

    @pl.loop(0, ROWS_SUB // CH)
    def _(t):
        pltpu.sync_copy(rows_v, acc.at[pl.ds(r0 + t * CH, CH)])

    plsc.subcore_barrier()
    base = wid * CH

    @pl.loop(0, NCHUNK)
    def _(j):
        off = base + j * (NW * CH)
        pltpu.sync_copy(src_hbm.at[pl.ds(off, CH)], src_v)
        pltpu.sync_copy(dst_hbm.at[pl.ds(off, CH)], dst_v)
        pltpu.sync_copy(hs_hbm.at[src_v], rows_v)
        pltpu.sync_copy(rows_v, acc.at[dst_v], add=True)

    plsc.subcore_barrier()
    pltpu.sync_copy(acc.at[pl.ds(r0, ROWS_SUB)],
                    p_hbm.at[c].at[pl.ds(r0, ROWS_SUB)])


BM = 640
BN = 400
_HI = lax.Precision.HIGHEST


def _mm_body(x_ref, w_ref, o_ref):
    o_ref[...] = jnp.dot(x_ref[...], w_ref[...],
                         preferred_element_type=f32, precision=_HI)


_mm = pl.pallas_call(
    _mm_body,
    grid=(R // BM,),
    in_specs=[pl.BlockSpec((BM, D), lambda i: (i, 0)),
              pl.BlockSpec((D, D), lambda i: (0, 0))],
    out_specs=pl.BlockSpec((BM, D), lambda i: (i, 0)),
    out_shape=jax.ShapeDtypeStruct((R, D), f32),
)


def _scale_body(h_ref, d_ref, o_ref):
    o_ref[...] = h_ref[...] * lax.rsqrt(d_ref[...])


_scale = pl.pallas_call(
    _scale_body,
    grid=(R // BM,),
    in_specs=[pl.BlockSpec((BM, D), lambda i: (i, 0)),
              pl.BlockSpec((BM, 1), lambda i: (i, 0))],
    out_specs=pl.BlockSpec((BM, D), lambda i: (i, 0)),
    out_shape=jax.ShapeDtypeStruct((R, D), f32),
)


def _combine1_body(p0_ref, p1_ref, hs_ref, d_ref, b_ref, w_ref,
                   h_ref, hs2_ref):
    dis = lax.rsqrt(d_ref[...])
    agg = p0_ref[0] + p1_ref[0] + hs_ref[...]
    e = jnp.maximum(dis * agg + b_ref[...], 0.0)
    h_ref[...] = e
    hs2_ref[...] = jnp.dot(e, w_ref[...],
                           preferred_element_type=f32, precision=_HI) * dis


_combine1 = pl.pallas_call(
    _combine1_body,
    grid=(R // BM,),
    in_specs=[pl.BlockSpec((1, BM, D), lambda i: (0, i, 0)),
              pl.BlockSpec((1, BM, D), lambda i: (1, i, 0)),
              pl.BlockSpec((BM, D), lambda i: (i, 0)),
              pl.BlockSpec((BM, 1), lambda i: (i, 0)),
              pl.BlockSpec((1, D), lambda i: (0, 0)),
              pl.BlockSpec((D, D), lambda i: (0, 0))],
    out_specs=[pl.BlockSpec((BM, D), lambda i: (i, 0)),
               pl.BlockSpec((BM, D), lambda i: (i, 0))],
    out_shape=[jax.ShapeDtypeStruct((R, D), f32),
               jax.ShapeDtypeStruct((R, D), f32)],
)


def _combine2_body(q0_ref, q1_ref, hs_ref, d_ref, b_ref, h1_ref, o_ref):
    dis = lax.rsqrt(d_ref[...])
    agg = q0_ref[0] + q1_ref[0] + hs_ref[...]
    h2 = jnp.maximum(dis * agg + b_ref[...], 0.0)
    o_ref[...] = jnp.concatenate([h1_ref[...], h2], axis=1)


_combine2 = pl.pallas_call(
    _combine2_body,
    grid=(N // BN,),
    in_specs=[pl.BlockSpec((1, BN, D), lambda i: (0, i, 0)),
              pl.BlockSpec((1, BN, D), lambda i: (1, i, 0)),
              pl.BlockSpec((BN, D), lambda i: (i, 0)),
              pl.BlockSpec((BN, 1), lambda i: (i, 0)),
              pl.BlockSpec((1, D), lambda i: (0, 0)),
              pl.BlockSpec((BN, D), lambda i: (i, 0))],
    out_specs=pl.BlockSpec((BN, 2 * D), lambda i: (i, 0)),
    out_shape=jax.ShapeDtypeStruct((N, 2 * D), f32),
)


def kernel(x, edge_index, W1, b1, W2, b2):
    x = x.astype(f32)
    src = edge_index[0].astype(i32)
    dst = edge_index[1].astype(i32)
    pad_idx = jnp.full((E_PAD - E,), N, i32)
    src_p = jnp.concatenate([src, pad_idx])
    dst_p = jnp.concatenate([dst, pad_idx])
    x_pad = jnp.concatenate([x, jnp.zeros((R - N, D), f32)])
    b1r = b1.reshape(1, D).astype(f32)
    b2r = b2.reshape(1, D).astype(f32)

    deg32 = _deg_kernel(dst_p)
    degp = deg32.reshape(NW, R).sum(axis=0).reshape(R, 1) + 1.0
    h1 = _mm(x_pad, W1.astype(f32))
    hs1 = _scale(h1, degp)
    p = _agg_kernel(hs1, src_p, dst_p)
    h1o, hs2 = _combine1(p, p, hs1, degp, b1r, W2.astype(f32))
    q = _agg_kernel(hs2, src_p, dst_p)
    return _combine2(q, q, hs2, degp, b2r, h1o)
